# Initial kernel scaffold; baseline (speedup 1.0000x reference)
import jax
import jax.numpy as jnp
from jax.experimental import pallas as pl


def kernel(observations):
    raise NotImplementedError("write your pallas kernel here")


# speedup vs baseline: 2.4147x; 2.4147x over previous
import functools

import numpy as np
import jax
import jax.numpy as jnp
from jax import lax
from jax.experimental import pallas as pl
from jax.experimental.pallas import tpu as pltpu
from jax.experimental.pallas import tpu_sc as plsc

_T = 8192
_PREV = 16
_POST = 16
_Y = 64
_W = _PREV + _POST + 1
_C = _Y * _W
_PT = _T + _PREV + _POST
_L = 16
_NV = _C // _L
_NC = 2
_NS = 16
_NWK = _NC * _NS
_RPW = _T // _NWK
_SUB = 16
_NSUB = _RPW // _SUB
_CHUNK = _RPW + _W - 1

_c = np.arange(_C)
_KFLAT = jnp.asarray(
    ((_c % _W) * _Y + (_c // _W)).reshape(_NV, _L).astype(np.int32))


def _pad_body(obs_ref, out_ref):
    obs = obs_ref[...]
    mean = jnp.mean(obs, axis=0, keepdims=True)
    out_ref[:_PREV, :] = jnp.broadcast_to(mean, (_PREV, _Y))
    out_ref[_PREV:_PREV + _T, :] = obs
    out_ref[_PREV + _T:, :] = jnp.broadcast_to(mean, (_POST, _Y))


def _pad(obs):
    return pl.pallas_call(
        _pad_body,
        out_shape=jax.ShapeDtypeStruct((_PT, _Y), jnp.float32),
    )(obs)


_mesh = plsc.VectorSubcoreMesh(core_axis_name="c", subcore_axis_name="s")


@functools.partial(
    pl.kernel,
    mesh=_mesh,
    compiler_params=pltpu.CompilerParams(needs_layout_passes=False),
    out_type=jax.ShapeDtypeStruct((_T, _C), jnp.float32),
    scratch_types=[
        pltpu.VMEM((_CHUNK * _Y,), jnp.float32),
        pltpu.VMEM((_NV, _L), jnp.int32),
        pltpu.VMEM((_SUB, _C), jnp.float32),
        pltpu.VMEM((_SUB, _C), jnp.float32),
        pltpu.SemaphoreType.DMA,
        pltpu.SemaphoreType.DMA,
    ],
)
def _window_gather(padded_hbm, kf_hbm, out_hbm,
                   chunk_v, kf_v, buf0, buf1, sem0, sem1):
    wid = lax.axis_index("s") * _NC + lax.axis_index("c")
    t0 = wid * _RPW
    pltpu.sync_copy(kf_hbm, kf_v)
    pltpu.sync_copy(padded_hbm.at[pl.ds(t0 * _Y, _CHUNK * _Y)], chunk_v)

    bufs = (buf0, buf1)
    sems = (sem0, sem1)
    pending = [None, None]
    for sub in range(_NSUB):
        b = sub % 2
        if pending[b] is not None:
            pending[b].wait()
        buf = bufs[b]
        base = sub * _SUB

        def j_body(j, carry, buf=buf, base=base):
            kf = kf_v[j]
            off = j * _L
            for t in range(_SUB):
                fidx = kf + (base + t) * _Y
                x = plsc.load_gather(chunk_v, [fidx])
                buf[t, pl.ds(off, _L)] = x
            return carry

        lax.fori_loop(0, _NV, j_body, 0)
        pending[b] = pltpu.async_copy(
            buf, out_hbm.at[pl.ds(t0 + base, _SUB)], sems[b])
    for p in pending:
        if p is not None:
            p.wait()


def kernel(observations):
    padded = _pad(observations)
    return _window_gather(padded.reshape(-1), _KFLAT)
